# baseline (device time: 8565 ns/iter reference)
import jax
import jax.numpy as jnp
from jax import lax
from jax.experimental import pallas as pl
from jax.experimental.pallas import tpu as pltpu

C = 64


def kernel(x, dest):
    m, n = x.shape
    maxc = m // C

    def body(
        x_hbm, dest_ref, out_ref, x_vmem, send_buf, recv_buf, send_sems,
        recv_sems, copy_sem,
    ):
        my_x = lax.axis_index("x")
        my_y = lax.axis_index("y")
        nbr = (my_x, 1 - my_y)

        recv_buf[...] = jnp.zeros((m, n), jnp.bfloat16)

        barrier_sem = pltpu.get_barrier_semaphore()
        pl.semaphore_signal(
            barrier_sem, inc=1, device_id=nbr, device_id_type=pl.DeviceIdType.MESH
        )

        cp = pltpu.make_async_copy(x_hbm, x_vmem, copy_sem)
        cp.start()

        d = dest_ref[...][None, :]
        mkeep = d == my_y
        mkeep_bf = mkeep.astype(jnp.bfloat16)

        row = lax.broadcasted_iota(jnp.int32, (m, m), 0)
        col = lax.broadcasted_iota(jnp.int32, (m, m), 1)
        tri = (row < col).astype(jnp.bfloat16)

        before_keep = jnp.dot(
            mkeep_bf, tri, preferred_element_type=jnp.float32
        ).astype(jnp.int32)
        kept = jnp.sum(mkeep_bf.astype(jnp.float32)).astype(jnp.int32)
        K = m - kept

        col1 = lax.broadcasted_iota(jnp.int32, (1, m), 1)
        before_send = col1 - before_keep
        not_keep = jnp.logical_not(mkeep)

        cp.wait()
        x_bf = x_vmem[...].astype(jnp.bfloat16)

        pl.semaphore_wait(barrier_sem, 1)

        rdmas = [
            pltpu.make_async_remote_copy(
                src_ref=send_buf.at[pl.ds(h * C, C)],
                dst_ref=recv_buf.at[pl.ds(h * C, C)],
                send_sem=send_sems.at[h],
                recv_sem=recv_sems.at[h],
                device_id=nbr,
                device_id_type=pl.DeviceIdType.MESH,
            )
            for h in range(maxc)
        ]

        for h in range(maxc):
            rowh = h * C + lax.broadcasted_iota(jnp.int32, (C, m), 0)
            p_send_h = ((rowh == before_send) & not_keep).astype(jnp.bfloat16)
            send_buf[pl.ds(h * C, C)] = jnp.dot(
                p_send_h, x_bf, preferred_element_type=jnp.float32
            ).astype(jnp.bfloat16)

            @pl.when(h * C < K)
            def _(h=h):
                rdmas[h].start()

        keep_off = jnp.where(my_y == 0, 0, K)
        recv_off = jnp.where(my_y == 0, kept, 0)
        p_keep = ((row == before_keep + keep_off) & mkeep).astype(jnp.bfloat16)
        acc = jnp.dot(p_keep, x_bf, preferred_element_type=jnp.float32)

        for h in range(maxc):

            @pl.when(h * C < K)
            def _(h=h):
                rdmas[h].wait_recv()

        shifted = pltpu.roll(recv_buf[...], recv_off, 0)
        out_ref[...] = (acc + shifted.astype(jnp.float32)).astype(jnp.bfloat16)

        for h in range(maxc):

            @pl.when(h * C < K)
            def _(h=h):
                rdmas[h].wait_send()

    return pl.pallas_call(
        body,
        out_shape=jax.ShapeDtypeStruct((m, n), jnp.bfloat16),
        in_specs=[
            pl.BlockSpec(memory_space=pl.ANY),
            pl.BlockSpec(memory_space=pltpu.VMEM),
        ],
        out_specs=pl.BlockSpec(memory_space=pltpu.VMEM),
        scratch_shapes=[
            pltpu.VMEM((m, n), jnp.float32),
            pltpu.VMEM((m, n), jnp.bfloat16),
            pltpu.VMEM((m, n), jnp.bfloat16),
            pltpu.SemaphoreType.DMA((maxc,)),
            pltpu.SemaphoreType.DMA((maxc,)),
            pltpu.SemaphoreType.DMA,
        ],
        compiler_params=pltpu.CompilerParams(collective_id=0),
    )(x, dest)
